# baseline (speedup 1.0000x reference)
import functools

import jax
import jax.numpy as jnp
from jax import lax
from jax.experimental import pallas as pl
from jax.experimental.pallas import tpu as pltpu
from jax.experimental.pallas import tpu_sc as plsc

_NC = 2
_NS = 16
_NW = _NC * _NS
_L = 16
_CH = 4


def _sc_gather_kernel(Q, D, NCHUNK, gidx_hbm, w_hbm, kv_hbm, out_hbm,
                      idx_v, w_v, buf_v, sem):
    wid = lax.axis_index("s") * _NC + lax.axis_index("c")
    pltpu.sync_copy(gidx_hbm.at[wid], idx_v)
    pltpu.sync_copy(w_hbm.at[wid], w_v)
    base_out = wid * Q

    full = NCHUNK - 1
    tail = Q - full * _CH

    def scale_row(c, j):
        wsp = w_v[c * _CH + j]

        def mul_body(s, _):
            off = s * (4 * _L)
            for u in range(4):
                sl = pl.ds(off + u * _L, _L)
                buf_v[j, sl] = buf_v[j, sl] * wsp
            return 0

        lax.fori_loop(0, D // (4 * _L), mul_body, 0)

    def chunk_body(c, _):
        pltpu.async_copy(kv_hbm.at[idx_v.at[c]], buf_v, sem).wait()
        for j in range(_CH):
            scale_row(c, j)
        for j in range(_CH):
            pltpu.sync_copy(
                buf_v.at[j],
                out_hbm.at[pl.ds((base_out + c * _CH + j) * D, D)])
        return 0

    lax.fori_loop(0, full, chunk_body, 0)
    pltpu.async_copy(kv_hbm.at[idx_v.at[full]], buf_v, sem).wait()
    for j in range(tail):
        scale_row(full, j)
    for j in range(tail):
        pltpu.sync_copy(
            buf_v.at[j],
            out_hbm.at[pl.ds((base_out + full * _CH + j) * D, D)])


def kernel(r_idx, r_weight, kv):
    n, p2, w2, c_kv = kv.shape
    topk = r_idx.shape[-1]
    R = n * p2
    D = w2 * c_kv
    nrows = R * topk
    assert nrows % _NW == 0
    Q = nrows // _NW
    nchunk = -(-Q // _CH)
    qp = 16 * (-(-nchunk * _CH // 16))
    qpad = qp - Q

    kv_flat = kv.reshape(R, D)
    gidx = (jnp.arange(n, dtype=jnp.int32)[:, None, None] * p2
            + r_idx).reshape(_NW, Q)
    w_all = r_weight.reshape(_NW, Q)
    gidx_p = jnp.pad(gidx, ((0, 0), (0, qpad))).reshape(_NW, qp // _CH, _CH)
    w_p = jnp.broadcast_to(
        jnp.pad(w_all, ((0, 0), (0, qpad)))[:, :, None], (_NW, qp, _L))

    mesh = plsc.VectorSubcoreMesh(core_axis_name="c", subcore_axis_name="s")
    body = functools.partial(_sc_gather_kernel, Q, D, nchunk)
    out = pl.kernel(
        body,
        out_type=jax.ShapeDtypeStruct((nrows * D,), jnp.float32),
        mesh=mesh,
        compiler_params=pltpu.CompilerParams(use_tc_tiling_on_sc=False),
        scratch_types=[
            pltpu.VMEM((qp // _CH, _CH), jnp.int32),
            pltpu.VMEM((qp, _L), jnp.float32),
            pltpu.VMEM((_CH, D), jnp.float32),
            pltpu.SemaphoreType.DMA,
        ],
    )(gidx_p, w_p, kv_flat)
    return out.reshape(n, p2, topk, w2, c_kv)


# speedup vs baseline: 1.0403x; 1.0403x over previous
import functools

import jax
import jax.numpy as jnp
from jax import lax
from jax.experimental import pallas as pl
from jax.experimental.pallas import tpu as pltpu
from jax.experimental.pallas import tpu_sc as plsc

_NC = 2
_NS = 16
_NW = _NC * _NS
_L = 16
_CH = 4


def _sc_gather_kernel(R, topk, w2, c_kv, p2, gidx_hbm, w_hbm, kv_hbm,
                      out_hbm, idx_v, w_v, buf_v, sem):
    wid = lax.axis_index("s") * _NC + lax.axis_index("c")
    nchunk = topk // _CH
    base = R // _NW
    extra = R - base * _NW
    npairs = base + jnp.where(wid < extra, 1, 0)

    def pair_body(k, _):
        r = wid + _NW * k
        b = r // p2
        i = r - b * p2
        pltpu.sync_copy(gidx_hbm.at[r], idx_v)
        pltpu.sync_copy(w_hbm.at[r], w_v)

        def chunk_body(c, _):
            pltpu.async_copy(kv_hbm.at[idx_v.at[c]], buf_v, sem).wait()
            for j in range(_CH):
                wsp = w_v[c * _CH + j]

                def mul_body(s, _, j=j, wsp=wsp):
                    for u in range(c_kv // _L):
                        sl = pl.ds(u * _L, _L)
                        buf_v[j, s, sl] = buf_v[j, s, sl] * wsp
                    return 0

                lax.fori_loop(0, w2, mul_body, 0)
            pltpu.sync_copy(buf_v, out_hbm.at[b, i, pl.ds(c * _CH, _CH)])
            return 0

        lax.fori_loop(0, nchunk, chunk_body, 0)
        return 0

    lax.fori_loop(0, npairs, pair_body, 0)


def kernel(r_idx, r_weight, kv):
    n, p2, w2, c_kv = kv.shape
    topk = r_idx.shape[-1]
    R = n * p2
    kv_tab = kv.reshape(R, w2, c_kv)
    pad = _L - topk
    gidx = (jnp.arange(n, dtype=jnp.int32)[:, None, None] * p2
            + r_idx).reshape(R, topk)
    gidx_p = jnp.pad(gidx, ((0, 0), (0, pad))).reshape(R, _L // _CH, _CH)
    w_p = jnp.broadcast_to(
        jnp.pad(r_weight.reshape(R, topk), ((0, 0), (0, pad)))[:, :, None],
        (R, _L, _L))

    mesh = plsc.VectorSubcoreMesh(core_axis_name="c", subcore_axis_name="s")
    body = functools.partial(_sc_gather_kernel, R, topk, w2, c_kv, p2)
    out = pl.kernel(
        body,
        out_type=jax.ShapeDtypeStruct((n, p2, topk, w2, c_kv), jnp.float32),
        mesh=mesh,
        compiler_params=pltpu.CompilerParams(use_tc_tiling_on_sc=False),
        scratch_types=[
            pltpu.VMEM((_L // _CH, _CH), jnp.int32),
            pltpu.VMEM((_L, _L), jnp.float32),
            pltpu.VMEM((_CH, w2, c_kv), jnp.float32),
            pltpu.SemaphoreType.DMA,
        ],
    )(gidx_p, w_p, kv_tab)
    return out


# speedup vs baseline: 1.4562x; 1.3998x over previous
import functools

import jax
import jax.numpy as jnp
from jax import lax
from jax.experimental import pallas as pl
from jax.experimental.pallas import tpu as pltpu
from jax.experimental.pallas import tpu_sc as plsc

_NC = 2
_NS = 16
_NW = _NC * _NS
_L = 16
_CH = 4


def _sc_gather_kernel(R, topk, w2, c_kv, p2, gidx_hbm, w_hbm, kv_hbm,
                      out_hbm, idx_v, w_v, buf_v, sem):
    wid = lax.axis_index("s") * _NC + lax.axis_index("c")
    nchunk = topk // _CH
    base = R // _NW
    extra = R - base * _NW
    npairs = base + jnp.where(wid < extra, 1, 0)

    def pair_body(k, _):
        r = wid + _NW * k
        b = r // p2
        i = r - b * p2
        pltpu.sync_copy(gidx_hbm.at[r], idx_v)
        pltpu.sync_copy(w_hbm.at[r], w_v)

        def chunk_body(c, _):
            pltpu.async_copy(kv_hbm.at[idx_v.at[c]], buf_v, sem).wait()
            for j in range(_CH):
                wsp = w_v[c * _CH + j]

                def mul_body(s, _, j=j, wsp=wsp):
                    for u in range(c_kv // _L):
                        sl = pl.ds(u * _L, _L)
                        buf_v[j, s, sl] = buf_v[j, s, sl] * wsp
                    return 0

                lax.fori_loop(0, w2, mul_body, 0)
            for j in range(_CH):
                pltpu.sync_copy(buf_v.at[j],
                                out_hbm.at[i, c * _CH + j, :, b, :])
            return 0

        lax.fori_loop(0, nchunk, chunk_body, 0)
        return 0

    lax.fori_loop(0, npairs, pair_body, 0)


def kernel(r_idx, r_weight, kv):
    n, p2, w2, c_kv = kv.shape
    topk = r_idx.shape[-1]
    R = n * p2
    kv_tab = kv.reshape(R, w2, c_kv)
    pad = _L - topk
    gidx = (jnp.arange(n, dtype=jnp.int32)[:, None, None] * p2
            + r_idx).reshape(R, topk)
    gidx_p = jnp.pad(gidx, ((0, 0), (0, pad))).reshape(R, _L // _CH, _CH)
    w_p = jnp.broadcast_to(
        jnp.pad(r_weight.reshape(R, topk), ((0, 0), (0, pad)))[:, :, None],
        (R, _L, _L))

    mesh = plsc.VectorSubcoreMesh(core_axis_name="c", subcore_axis_name="s")
    body = functools.partial(_sc_gather_kernel, R, topk, w2, c_kv, p2)
    out = pl.kernel(
        body,
        out_type=jax.ShapeDtypeStruct((p2, topk, w2, n, c_kv), jnp.float32),
        mesh=mesh,
        compiler_params=pltpu.CompilerParams(use_tc_tiling_on_sc=False),
        scratch_types=[
            pltpu.VMEM((_L // _CH, _CH), jnp.int32),
            pltpu.VMEM((_L, _L), jnp.float32),
            pltpu.VMEM((_CH, w2, c_kv), jnp.float32),
            pltpu.SemaphoreType.DMA,
        ],
    )(gidx_p, w_p, kv_tab)
    return jnp.transpose(out, (3, 0, 1, 2, 4))


# speedup vs baseline: 1.7305x; 1.1884x over previous
import functools

import jax
import jax.numpy as jnp
from jax import lax
from jax.experimental import pallas as pl
from jax.experimental.pallas import tpu as pltpu
from jax.experimental.pallas import tpu_sc as plsc

_NC = 2
_NS = 16
_NW = _NC * _NS
_L = 16
_CH = 4


def _sc_gather_kernel(nch, nreal, p2, topk, w2, c_kv, gidx_hbm, w_hbm,
                      kv_hbm, out_hbm, dump_hbm, idx_v, w_v, buf0, buf1,
                      gsem0, gsem1, wsem0, wsem1):
    wid = lax.axis_index("s") * _NC + lax.axis_index("c")
    pltpu.sync_copy(gidx_hbm.at[wid], idx_v)
    pltpu.sync_copy(w_hbm.at[wid], w_v)

    bufs = (buf0, buf1)
    gsems = (gsem0, gsem1)
    wsems = (wsem0, wsem1)
    nchw = topk // _CH

    def start_gather(q):
        return pltpu.async_copy(
            kv_hbm.at[idx_v.at[q]], bufs[q % 2], gsems[q % 2])

    def scale(q):
        buf = bufs[q % 2]
        wsp = [w_v[q, j] for j in range(_CH)]

        def mul_body(s, _):
            for j in range(_CH):
                for u in range(c_kv // _L):
                    sl = pl.ds(u * _L, _L)
                    buf[j, s, sl] = buf[j, s, sl] * wsp[j]
            return 0

        lax.fori_loop(0, w2, mul_body, 0)

    def start_writes(q):
        buf = bufs[q % 2]
        wsem = wsems[q % 2]
        h = wid + _NW * q
        r = h // nchw
        c = h - nchw * r
        b = r // p2
        i = r - p2 * b
        cps = []
        for j in range(_CH):
            dst = out_hbm.at[i, c * _CH + j, :, b, :]
            if q == nch - 1:
                ok = h < nreal

                @pl.when(ok)
                def _(dst=dst, j=j):
                    pltpu.async_copy(buf.at[j], dst, wsem)

                @pl.when(jnp.logical_not(ok))
                def _(j=j):
                    pltpu.async_copy(buf.at[j], dump_hbm.at[j], wsem)

                cps.append(pltpu.make_async_copy(buf.at[j], dump_hbm.at[j],
                                                 wsem))
            else:
                cps.append(pltpu.async_copy(buf.at[j], dst, wsem))
        return cps

    writes = {}
    gathers = {0: start_gather(0)}
    for q in range(nch):
        gathers.pop(q).wait()
        if q + 1 < nch:
            if q >= 1:
                for cp in writes.pop(q - 1):
                    cp.wait()
            gathers[q + 1] = start_gather(q + 1)
        scale(q)
        writes[q] = start_writes(q)
    for q in sorted(writes):
        for cp in writes.pop(q):
            cp.wait()


def kernel(r_idx, r_weight, kv):
    n, p2, w2, c_kv = kv.shape
    topk = r_idx.shape[-1]
    R = n * p2
    kv_tab = kv.reshape(R, w2, c_kv)
    nchunks = R * topk // _CH
    nch = -(-nchunks // _NW)

    gidx = (jnp.arange(n, dtype=jnp.int32)[:, None, None] * p2
            + r_idx).reshape(nchunks, _CH)
    wflat = r_weight.reshape(nchunks, _CH)
    padc = nch * _NW - nchunks
    gidx_p = jnp.pad(gidx, ((0, padc), (0, 0))).reshape(nch, _NW, _CH)
    gidx_w = jnp.transpose(gidx_p, (1, 0, 2))
    qpad = 8 * (-(-nch // 8)) - nch
    gidx_w = jnp.pad(gidx_w, ((0, 0), (0, qpad), (0, 0)))
    w_p = jnp.pad(wflat, ((0, padc), (0, 0))).reshape(nch, _NW, _CH)
    w_w = jnp.pad(jnp.transpose(w_p, (1, 0, 2)), ((0, 0), (0, qpad), (0, 0)))
    w_w = jnp.broadcast_to(w_w[:, :, :, None], (_NW, nch + qpad, _CH, _L))

    mesh = plsc.VectorSubcoreMesh(core_axis_name="c", subcore_axis_name="s")
    body = functools.partial(_sc_gather_kernel, nch, nchunks, p2, topk, w2,
                             c_kv)
    out, _ = pl.kernel(
        body,
        out_type=(
            jax.ShapeDtypeStruct((p2, topk, w2, n, c_kv), jnp.float32),
            jax.ShapeDtypeStruct((_CH, w2, c_kv), jnp.float32),
        ),
        mesh=mesh,
        compiler_params=pltpu.CompilerParams(use_tc_tiling_on_sc=False),
        scratch_types=[
            pltpu.VMEM((nch + qpad, _CH), jnp.int32),
            pltpu.VMEM((nch + qpad, _CH, _L), jnp.float32),
            pltpu.VMEM((_CH, w2, c_kv), jnp.float32),
            pltpu.VMEM((_CH, w2, c_kv), jnp.float32),
            pltpu.SemaphoreType.DMA,
            pltpu.SemaphoreType.DMA,
            pltpu.SemaphoreType.DMA,
            pltpu.SemaphoreType.DMA,
        ],
    )(gidx_w, w_w, kv_tab)
    return jnp.transpose(out, (3, 0, 1, 2, 4))
